# baseline (speedup 1.0000x reference)
import functools

import jax
import jax.numpy as jnp
from jax import lax
from jax.experimental import pallas as pl
from jax.experimental.pallas import tpu as pltpu
from jax.experimental.pallas import tpu_sc as plsc

_NC = 2
_NS = 16
_NW = _NC * _NS
_BPW = 128
_LANES = 16


@functools.lru_cache(maxsize=None)
def _make_gather(seq, word_dim):
    assert word_dim % 8 == 0 and seq % 2 == 0 and seq >= 6
    dq = word_dim // 8
    mesh = plsc.VectorSubcoreMesh(core_axis_name="c", subcore_axis_name="s")

    @functools.partial(
        pl.kernel,
        mesh=mesh,
        compiler_params=pltpu.CompilerParams(
            use_tc_tiling_on_sc=False, needs_layout_passes=False),
        out_type=jax.ShapeDtypeStruct((seq, dq, _NW, 8 * _BPW), jnp.float32),
        scratch_types=[
            pltpu.VMEM((seq, _BPW), jnp.int32),
            pltpu.VMEM((_BPW, word_dim), jnp.float32),
            pltpu.VMEM((_BPW, word_dim), jnp.float32),
            pltpu.VMEM((dq, 8 * _BPW), jnp.float32),
            pltpu.VMEM((dq, 8 * _BPW), jnp.float32),
            pltpu.SemaphoreType.DMA,
            pltpu.SemaphoreType.DMA,
            pltpu.SemaphoreType.DMA,
            pltpu.SemaphoreType.DMA,
        ],
    )
    def gather_kernel(table_hbm, idx_hbm, out_hbm,
                      idx_v, g0, g1, t0, t1, gs0, gs1, os0, os1):
        wid = lax.axis_index("s") * _NC + lax.axis_index("c")
        pltpu.sync_copy(idx_hbm.at[wid], idx_v)
        gbuf = (g0, g1)
        tbuf = (t0, t1)
        gsem = (gs0, gs1)
        osem = (os0, os1)

        def g_start(s, b):
            pltpu.async_copy(table_hbm.at[idx_v.at[s]], gbuf[b], gsem[b])

        def g_wait(b):
            pltpu.make_async_copy(
                table_hbm.at[idx_v.at[0]], gbuf[b], gsem[b]).wait()

        def w_start(s, b):
            pltpu.async_copy(tbuf[b], out_hbm.at[s, :, wid], osem[b])

        def w_wait(b):
            pltpu.make_async_copy(
                tbuf[b], out_hbm.at[0, :, wid], osem[b]).wait()

        iota = lax.iota(jnp.int32, _LANES)

        def transpose(b):
            g, t = gbuf[b], tbuf[b]
            for d in range(word_dim):
                cols = jnp.full((_LANES,), d, jnp.int32)
                for grp in range(_BPW // _LANES):
                    rows = iota + (grp * _LANES)
                    vec = plsc.load_gather(g, [rows, cols])
                    t[d // 8, pl.ds((d % 8) * _BPW + grp * _LANES, _LANES)] = vec

        g_start(0, 0)
        g_start(1, 1)
        for s in range(2):
            g_wait(s)
            transpose(s)
            w_start(s, s)
            g_start(s + 2, s)

        def body(i, carry):
            s0 = 2 + i * 2
            for b in range(2):
                s = s0 + b
                g_wait(b)
                w_wait(b)
                transpose(b)
                w_start(s, b)
                g_start(s + 2, b)
            return carry

        lax.fori_loop(0, (seq - 4) // 2, body, 0)

        for b in range(2):
            g_wait(b)
            w_wait(b)
            transpose(b)
            w_start(seq - 2 + b, b)
        w_wait(0)
        w_wait(1)

    return gather_kernel


def kernel(sent, W):
    batch, seq = sent.shape
    word_dim = W.shape[1]
    assert batch == _NW * _BPW
    idx = sent.astype(jnp.int32).reshape(_NW, _BPW, seq).transpose(0, 2, 1)
    y = _make_gather(seq, word_dim)(W, idx)
    y = y.reshape(seq, word_dim // 8, _NW, 8, _BPW)
    return y.transpose(2, 4, 0, 1, 3).reshape(batch, seq, word_dim)


# speedup vs baseline: 1.9720x; 1.9720x over previous
import functools

import jax
import jax.numpy as jnp
from jax import lax
from jax.experimental import pallas as pl
from jax.experimental.pallas import tpu as pltpu
from jax.experimental.pallas import tpu_sc as plsc

_NC = 2
_NS = 16
_NW = _NC * _NS
_BPW = 128
_LANES = 16


@functools.lru_cache(maxsize=None)
def _make_gather(seq, word_dim):
    assert word_dim % 8 == 0 and word_dim & (word_dim - 1) == 0
    assert seq % 2 == 0 and seq >= 6
    dq = word_dim // 8
    mesh = plsc.VectorSubcoreMesh(core_axis_name="c", subcore_axis_name="s")

    @functools.partial(
        pl.kernel,
        mesh=mesh,
        compiler_params=pltpu.CompilerParams(
            use_tc_tiling_on_sc=False, needs_layout_passes=False),
        out_type=jax.ShapeDtypeStruct((seq, dq, _NW, 8 * _BPW), jnp.float32),
        scratch_types=[
            pltpu.VMEM((seq, _BPW), jnp.int32),
            pltpu.VMEM((_BPW, word_dim), jnp.float32),
            pltpu.VMEM((_BPW, word_dim), jnp.float32),
            pltpu.VMEM((dq, 8 * _BPW), jnp.float32),
            pltpu.VMEM((dq, 8 * _BPW), jnp.float32),
            pltpu.SemaphoreType.DMA,
            pltpu.SemaphoreType.DMA,
            pltpu.SemaphoreType.DMA,
            pltpu.SemaphoreType.DMA,
        ],
    )
    def gather_kernel(table_hbm, idx_hbm, out_hbm,
                      idx_v, g0, g1, t0, t1, gs0, gs1, os0, os1):
        wid = lax.axis_index("s") * _NC + lax.axis_index("c")
        pltpu.sync_copy(idx_hbm.at[wid], idx_v)
        gbuf = (g0, g1)
        tbuf = (t0, t1)
        gsem = (gs0, gs1)
        osem = (os0, os1)

        def g_start(s, b):
            pltpu.async_copy(table_hbm.at[idx_v.at[s]], gbuf[b], gsem[b])

        def g_wait(b):
            pltpu.make_async_copy(
                table_hbm.at[idx_v.at[0]], gbuf[b], gsem[b]).wait()

        def w_start(s, b):
            pltpu.async_copy(tbuf[b], out_hbm.at[s, :, wid], osem[b])

        def w_wait(b):
            pltpu.make_async_copy(
                tbuf[b], out_hbm.at[0, :, wid], osem[b]).wait()

        iota = lax.iota(jnp.int32, _LANES)
        rows_c = [iota + (r0 * _LANES) for r0 in range(_BPW // _LANES)]

        def transpose(b):
            g, t = gbuf[b], tbuf[b]

            def dbody(d0, c):
                dvec = (d0 + iota) & (word_dim - 1)
                trow = dvec >> 3
                tcolb = (dvec & 7) << 7
                for r0 in range(_BPW // _LANES):
                    rv = rows_c[r0]
                    vec = plsc.load_gather(g, [rv, dvec])
                    plsc.store_scatter(t, [trow, tcolb + rv], vec)
                return c

            lax.fori_loop(0, word_dim, dbody, 0)

        g_start(0, 0)
        g_start(1, 1)

        def body(i, carry):
            s0 = i * 2
            for b in range(2):
                s = s0 + b
                g_wait(b)
                @pl.when(s0 >= 2)
                def _():
                    w_wait(b)
                transpose(b)
                w_start(s, b)
                @pl.when(s0 + 2 < seq)
                def _():
                    g_start(s + 2, b)
            return carry

        lax.fori_loop(0, seq // 2, body, 0)
        w_wait(0)
        w_wait(1)

    return gather_kernel


def kernel(sent, W):
    batch, seq = sent.shape
    word_dim = W.shape[1]
    assert batch == _NW * _BPW
    idx = sent.astype(jnp.int32).reshape(_NW, _BPW, seq).transpose(0, 2, 1)
    y = _make_gather(seq, word_dim)(W, idx)
    y = y.reshape(seq, word_dim // 8, _NW, 8, _BPW)
    return y.transpose(2, 4, 0, 1, 3).reshape(batch, seq, word_dim)
